# baseline (speedup 1.0000x reference)
import functools

import jax
import jax.numpy as jnp
from jax import lax
from jax.experimental import pallas as pl
from jax.experimental.pallas import tpu as pltpu
from jax.experimental.pallas import tpu_sc as plsc

B = 16384
D = 16


def _tc_transpose(tab_t, cb=8192):
    n = tab_t.shape[1]
    grid = (n + cb - 1) // cb

    def body(in_r, out_r):
        out_r[...] = in_r[...].T

    return pl.pallas_call(
        body,
        grid=(grid,),
        in_specs=[pl.BlockSpec((D, cb), lambda i: (0, i))],
        out_specs=pl.BlockSpec((cb, D), lambda i: (i, 0)),
        out_shape=jax.ShapeDtypeStruct((n, D), jnp.float32),
    )(tab_t)


def _sc_gather4(uidx, iidx, ueg, ieg, uem, iem):
    info = plsc.get_sparse_core_info()
    nw = info.num_cores * info.num_subcores
    bpw = B // nw
    mesh = plsc.VectorSubcoreMesh(core_axis_name="c", subcore_axis_name="s")

    @functools.partial(
        pl.kernel,
        mesh=mesh,
        out_type=[jax.ShapeDtypeStruct((B, D), jnp.float32)] * 4,
        scratch_types=[
            pltpu.VMEM((bpw,), jnp.int32),
            pltpu.VMEM((bpw,), jnp.int32),
            pltpu.VMEM((bpw, D), jnp.float32),
            pltpu.VMEM((bpw, D), jnp.float32),
            pltpu.VMEM((bpw, D), jnp.float32),
            pltpu.VMEM((bpw, D), jnp.float32),
            pltpu.SemaphoreType.DMA,
        ],
        compiler_params=pltpu.CompilerParams(use_tc_tiling_on_sc=False),
    )
    def k(uidx_hbm, iidx_hbm, ueg_hbm, ieg_hbm, uem_hbm, iem_hbm,
          oug, oig, oum, oim, uv, iv, r0, r1, r2, r3, sem):
        wid = lax.axis_index("s") * info.num_cores + lax.axis_index("c")
        base = wid * bpw
        pltpu.sync_copy(uidx_hbm.at[pl.ds(base, bpw)], uv)
        pltpu.sync_copy(iidx_hbm.at[pl.ds(base, bpw)], iv)
        c0 = pltpu.async_copy(ueg_hbm.at[uv], r0, sem)
        c1 = pltpu.async_copy(ieg_hbm.at[iv], r1, sem)
        c2 = pltpu.async_copy(uem_hbm.at[uv], r2, sem)
        c3 = pltpu.async_copy(iem_hbm.at[iv], r3, sem)
        c0.wait()
        c1.wait()
        c2.wait()
        c3.wait()
        pltpu.sync_copy(r0, oug.at[pl.ds(base, bpw)])
        pltpu.sync_copy(r1, oig.at[pl.ds(base, bpw)])
        pltpu.sync_copy(r2, oum.at[pl.ds(base, bpw)])
        pltpu.sync_copy(r3, oim.at[pl.ds(base, bpw)])

    return k(uidx, iidx, ueg, ieg, uem, iem)


def _tc_mlp_body(ug_r, ig_r, um_r, im_r, w0a_r, w0b_r, b0_r, w1_r, b1_r,
                 w2_r, b2_r, w3_r, b3_r, wpg_r, wph_r, bp_r, out_r):
    f32 = jnp.float32
    gmf = ug_r[...] * ig_r[...]
    h = jnp.dot(um_r[...], w0a_r[...], preferred_element_type=f32)
    h = h + jnp.dot(im_r[...], w0b_r[...], preferred_element_type=f32)
    h = jnp.maximum(h + b0_r[...], 0.0)
    h = jnp.maximum(jnp.dot(h, w1_r[...], preferred_element_type=f32) + b1_r[...], 0.0)
    h = jnp.maximum(jnp.dot(h, w2_r[...], preferred_element_type=f32) + b2_r[...], 0.0)
    h = jnp.maximum(jnp.dot(h, w3_r[...], preferred_element_type=f32) + b3_r[...], 0.0)
    pred = jnp.dot(gmf, wpg_r[...], preferred_element_type=f32)
    pred = pred + jnp.dot(h, wph_r[...], preferred_element_type=f32)
    out_r[...] = pred + bp_r[...]


def _tc_mlp(ug, ig, um, im, w0a, w0b, b0, w1t, b1, w2t, b2, w3t, b3,
            wpg, wph, bp2):
    nblk = 8
    rb = B // nblk
    row_spec = pl.BlockSpec((rb, D), lambda i: (i, 0))

    def full(x):
        return pl.BlockSpec(x.shape, lambda i: (0,) * x.ndim)

    return pl.pallas_call(
        _tc_mlp_body,
        grid=(nblk,),
        in_specs=[row_spec, row_spec, row_spec, row_spec,
                  full(w0a), full(w0b), full(b0), full(w1t), full(b1),
                  full(w2t), full(b2), full(w3t), full(b3),
                  full(wpg), full(wph), full(bp2)],
        out_specs=pl.BlockSpec((rb, 1), lambda i: (i, 0)),
        out_shape=jax.ShapeDtypeStruct((B, 1), jnp.float32),
    )(ug, ig, um, im, w0a, w0b, b0, w1t, b1, w2t, b2, w3t, b3, wpg, wph, bp2)


def kernel(user_indices, item_indices, user_embed_gmf, item_embed_gmf,
           user_embed_mlp, item_embed_mlp,
           W0, b0, W1, b1, W2, b2, W3, b3, Wp, bp):
    uidx = user_indices.astype(jnp.int32)
    iidx = item_indices.astype(jnp.int32)

    ueg = _tc_transpose(user_embed_gmf.T)
    ieg = _tc_transpose(item_embed_gmf.T)
    uem = _tc_transpose(user_embed_mlp.T)
    iem = _tc_transpose(item_embed_mlp.T)

    ug, ig, um, im = _sc_gather4(uidx, iidx, ueg, ieg, uem, iem)

    w0a = W0[:, :D].T
    w0b = W0[:, D:].T
    wpg = Wp[:, :D].T
    wph = Wp[:, D:].T
    pred = _tc_mlp(ug, ig, um, im, w0a, w0b, b0.reshape(1, -1),
                   W1.T, b1.reshape(1, -1), W2.T, b2.reshape(1, -1),
                   W3.T, b3.reshape(1, -1), wpg, wph, bp.reshape(1, 1))
    return jnp.squeeze(pred, axis=-1)


# speedup vs baseline: 1.8462x; 1.8462x over previous
import functools

import jax
import jax.numpy as jnp
from jax import lax
from jax.experimental import pallas as pl
from jax.experimental.pallas import tpu as pltpu
from jax.experimental.pallas import tpu_sc as plsc

B = 16384
D = 16
CH = 128


CB = 16384
PB = CB // 8


def _tc_pack(tab_t):
    n = tab_t.shape[1]
    grid = (n + CB - 1) // CB

    def body(in_r, out_r):
        for k in range(8):
            out_r[:, k * D:(k + 1) * D] = in_r[:, k * PB:(k + 1) * PB].T

    return pl.pallas_call(
        body,
        grid=(grid,),
        in_specs=[pl.BlockSpec((D, CB), lambda i: (0, i))],
        out_specs=pl.BlockSpec((PB, 128), lambda i: (i, 0)),
        out_shape=jax.ShapeDtypeStruct((grid * PB, 128), jnp.float32),
    )(tab_t)


def _sc_gather4(urow, usub, irow, isub, tug, tig, tum, tim):
    info = plsc.get_sparse_core_info()
    nw = info.num_cores * info.num_subcores
    bpw = B // nw
    nch = bpw // CH
    mesh = plsc.VectorSubcoreMesh(core_axis_name="c", subcore_axis_name="s")

    @functools.partial(
        pl.kernel,
        mesh=mesh,
        out_type=[jax.ShapeDtypeStruct((B, D), jnp.float32)] * 4,
        scratch_types=[
            pltpu.VMEM((bpw,), jnp.int32),
            pltpu.VMEM((bpw,), jnp.int32),
            pltpu.VMEM((bpw,), jnp.int32),
            pltpu.VMEM((bpw,), jnp.int32),
            pltpu.VMEM((CH, 128), jnp.float32),
            pltpu.VMEM((CH, 128), jnp.float32),
            pltpu.VMEM((bpw, D), jnp.float32),
            pltpu.VMEM((bpw, D), jnp.float32),
            pltpu.VMEM((bpw, D), jnp.float32),
            pltpu.VMEM((bpw, D), jnp.float32),
            pltpu.SemaphoreType.DMA,
            pltpu.SemaphoreType.DMA,
        ],
        compiler_params=pltpu.CompilerParams(
            use_tc_tiling_on_sc=False, needs_layout_passes=False),
    )
    def k(urow_h, usub_h, irow_h, isub_h, tug_h, tig_h, tum_h, tim_h,
          oug, oig, oum, oim, urv, usv, irv, isv, gb0, gb1,
          pug, pig, pum, pim, sem0, sem1):
        wid = lax.axis_index("s") * info.num_cores + lax.axis_index("c")
        base = wid * bpw
        pltpu.sync_copy(urow_h.at[pl.ds(base, bpw)], urv)
        pltpu.sync_copy(usub_h.at[pl.ds(base, bpw)], usv)
        pltpu.sync_copy(irow_h.at[pl.ds(base, bpw)], irv)
        pltpu.sync_copy(isub_h.at[pl.ds(base, bpw)], isv)

        steps = []
        for tab, rv, sv, ov in [(tug_h, urv, usv, pug),
                                (tig_h, irv, isv, pig),
                                (tum_h, urv, usv, pum),
                                (tim_h, irv, isv, pim)]:
            for c in range(nch):
                steps.append((tab, rv, sv, ov, c))

        gbufs = (gb0, gb1)
        sems = (sem0, sem1)
        iota16 = lax.iota(jnp.int32, 16)

        def issue(s):
            tab, rv, _, _, c = steps[s]
            return pltpu.async_copy(tab.at[rv.at[pl.ds(c * CH, CH)]],
                                    gbufs[s % 2], sems[s % 2])

        def extract(s, cp):
            _, _, sv, ov, c = steps[s]
            gb = gbufs[s % 2]
            cp.wait()

            def body(g, _):
                offs = sv[pl.ds(c * CH + g * 16, 16)]
                for jj in range(16):
                    lanei = jnp.broadcast_to(offs[jj], (16,)) + iota16
                    rowi = jnp.broadcast_to(g * 16 + jj, (16,))
                    vals = plsc.load_gather(gb, [rowi, lanei])
                    ov[c * CH + g * 16 + jj, :] = vals
                return 0

            lax.fori_loop(0, CH // 16, body, 0)

        cp = issue(0)
        for s in range(len(steps)):
            nxt = issue(s + 1) if s + 1 < len(steps) else None
            extract(s, cp)
            cp = nxt

        pltpu.sync_copy(pug, oug.at[pl.ds(base, bpw)])
        pltpu.sync_copy(pig, oig.at[pl.ds(base, bpw)])
        pltpu.sync_copy(pum, oum.at[pl.ds(base, bpw)])
        pltpu.sync_copy(pim, oim.at[pl.ds(base, bpw)])

    return k(urow, usub, irow, isub, tug, tig, tum, tim)


def _tc_mlp_body(ug_r, ig_r, um_r, im_r, w0a_r, w0b_r, b0_r, w1_r, b1_r,
                 w2_r, b2_r, w3_r, b3_r, wpg_r, wph_r, bp_r, out_r):
    f32 = jnp.float32
    gmf = ug_r[...] * ig_r[...]
    h = jnp.dot(um_r[...], w0a_r[...], preferred_element_type=f32)
    h = h + jnp.dot(im_r[...], w0b_r[...], preferred_element_type=f32)
    h = jnp.maximum(h + b0_r[...], 0.0)
    h = jnp.maximum(jnp.dot(h, w1_r[...], preferred_element_type=f32) + b1_r[...], 0.0)
    h = jnp.maximum(jnp.dot(h, w2_r[...], preferred_element_type=f32) + b2_r[...], 0.0)
    h = jnp.maximum(jnp.dot(h, w3_r[...], preferred_element_type=f32) + b3_r[...], 0.0)
    pred = jnp.dot(gmf, wpg_r[...], preferred_element_type=f32)
    pred = pred + jnp.dot(h, wph_r[...], preferred_element_type=f32)
    out_r[...] = pred + bp_r[...]


def _tc_mlp(ug, ig, um, im, w0a, w0b, b0, w1t, b1, w2t, b2, w3t, b3,
            wpg, wph, bp2):
    nblk = 8
    rb = B // nblk
    row_spec = pl.BlockSpec((rb, D), lambda i: (i, 0))

    def full(x):
        return pl.BlockSpec(x.shape, lambda i: (0,) * x.ndim)

    return pl.pallas_call(
        _tc_mlp_body,
        grid=(nblk,),
        in_specs=[row_spec, row_spec, row_spec, row_spec,
                  full(w0a), full(w0b), full(b0), full(w1t), full(b1),
                  full(w2t), full(b2), full(w3t), full(b3),
                  full(wpg), full(wph), full(bp2)],
        out_specs=pl.BlockSpec((rb, 1), lambda i: (i, 0)),
        out_shape=jax.ShapeDtypeStruct((B, 1), jnp.float32),
    )(ug, ig, um, im, w0a, w0b, b0, w1t, b1, w2t, b2, w3t, b3, wpg, wph, bp2)


def kernel(user_indices, item_indices, user_embed_gmf, item_embed_gmf,
           user_embed_mlp, item_embed_mlp,
           W0, b0, W1, b1, W2, b2, W3, b3, Wp, bp):
    uidx = user_indices.astype(jnp.int32)
    iidx = item_indices.astype(jnp.int32)

    def coords(idx):
        rem = idx & (CB - 1)
        row = (idx >> 14) * PB + (rem & (PB - 1))
        sub = ((rem >> 11) & 7) << 4
        return row, sub

    urow, usub = coords(uidx)
    irow, isub = coords(iidx)

    tug = _tc_pack(user_embed_gmf.T)
    tig = _tc_pack(item_embed_gmf.T)
    tum = _tc_pack(user_embed_mlp.T)
    tim = _tc_pack(item_embed_mlp.T)

    ug, ig, um, im = _sc_gather4(urow, usub, irow, isub, tug, tig, tum, tim)

    w0a = W0[:, :D].T
    w0b = W0[:, D:].T
    wpg = Wp[:, :D].T
    wph = Wp[:, D:].T
    pred = _tc_mlp(ug, ig, um, im, w0a, w0b, b0.reshape(1, -1),
                   W1.T, b1.reshape(1, -1), W2.T, b2.reshape(1, -1),
                   W3.T, b3.reshape(1, -1), wpg, wph, bp.reshape(1, 1))
    return jnp.squeeze(pred, axis=-1)


# speedup vs baseline: 5.0570x; 2.7391x over previous
import functools

import jax
import jax.numpy as jnp
from jax import lax
from jax.experimental import pallas as pl
from jax.experimental.pallas import tpu as pltpu
from jax.experimental.pallas import tpu_sc as plsc

B = 16384
D = 16
CH = 128


CB = 8192
PB = CB // 8


def _tc_pack2(tab_a, tab_b):
    n = tab_a.shape[1]
    grid = (n + CB - 1) // CB

    def body(ina_r, inb_r, outa_r, outb_r):
        for in_r, out_r in ((ina_r, outa_r), (inb_r, outb_r)):
            x = in_r[...]
            stacked = jnp.concatenate(
                [x[:, k * PB:(k + 1) * PB] for k in range(8)], axis=0)
            out_r[...] = stacked.T

    spec_in = pl.BlockSpec((D, CB), lambda i: (0, i))
    spec_out = pl.BlockSpec((PB, 128), lambda i: (i, 0))
    return pl.pallas_call(
        body,
        grid=(grid,),
        in_specs=[spec_in, spec_in],
        out_specs=[spec_out, spec_out],
        out_shape=[jax.ShapeDtypeStruct((grid * PB, 128), jnp.float32)] * 2,
    )(tab_a, tab_b)


def _sc_gather4(urow, usub, irow, isub, tug, tig, tum, tim):
    info = plsc.get_sparse_core_info()
    nw = info.num_cores * info.num_subcores
    bpw = B // nw
    nch = bpw // CH
    mesh = plsc.VectorSubcoreMesh(core_axis_name="c", subcore_axis_name="s")

    @functools.partial(
        pl.kernel,
        mesh=mesh,
        out_type=[jax.ShapeDtypeStruct((B, D), jnp.float32)] * 4,
        scratch_types=[
            pltpu.VMEM((bpw,), jnp.int32),
            pltpu.VMEM((bpw,), jnp.int32),
            pltpu.VMEM((bpw,), jnp.int32),
            pltpu.VMEM((bpw,), jnp.int32),
            pltpu.VMEM((CH, 128), jnp.float32),
            pltpu.VMEM((CH, 128), jnp.float32),
            pltpu.VMEM((bpw, D), jnp.float32),
            pltpu.VMEM((bpw, D), jnp.float32),
            pltpu.VMEM((bpw, D), jnp.float32),
            pltpu.VMEM((bpw, D), jnp.float32),
            pltpu.SemaphoreType.DMA,
            pltpu.SemaphoreType.DMA,
        ],
        compiler_params=pltpu.CompilerParams(
            use_tc_tiling_on_sc=False, needs_layout_passes=False),
    )
    def k(urow_h, usub_h, irow_h, isub_h, tug_h, tig_h, tum_h, tim_h,
          oug, oig, oum, oim, urv, usv, irv, isv, gb0, gb1,
          pug, pig, pum, pim, sem0, sem1):
        wid = lax.axis_index("s") * info.num_cores + lax.axis_index("c")
        base = wid * bpw
        pltpu.sync_copy(urow_h.at[pl.ds(base, bpw)], urv)
        pltpu.sync_copy(usub_h.at[pl.ds(base, bpw)], usv)
        pltpu.sync_copy(irow_h.at[pl.ds(base, bpw)], irv)
        pltpu.sync_copy(isub_h.at[pl.ds(base, bpw)], isv)

        steps = []
        for tab, rv, sv, ov in [(tug_h, urv, usv, pug),
                                (tig_h, irv, isv, pig),
                                (tum_h, urv, usv, pum),
                                (tim_h, irv, isv, pim)]:
            for c in range(nch):
                steps.append((tab, rv, sv, ov, c))

        gbufs = (gb0, gb1)
        sems = (sem0, sem1)
        iota16 = lax.iota(jnp.int32, 16)

        def issue(s):
            tab, rv, _, _, c = steps[s]
            return pltpu.async_copy(tab.at[rv.at[pl.ds(c * CH, CH)]],
                                    gbufs[s % 2], sems[s % 2])

        def extract(s, cp):
            _, _, sv, ov, c = steps[s]
            gb = gbufs[s % 2]
            cp.wait()

            def body(g, _):
                offs = sv[pl.ds(c * CH + g * 16, 16)]
                for jj in range(16):
                    lanei = jnp.broadcast_to(offs[jj], (16,)) + iota16
                    rowi = jnp.broadcast_to(g * 16 + jj, (16,))
                    vals = plsc.load_gather(gb, [rowi, lanei])
                    ov[c * CH + g * 16 + jj, :] = vals
                return 0

            lax.fori_loop(0, CH // 16, body, 0)

        cp = issue(0)
        for s in range(len(steps)):
            nxt = issue(s + 1) if s + 1 < len(steps) else None
            extract(s, cp)
            cp = nxt

        pltpu.sync_copy(pug, oug.at[pl.ds(base, bpw)])
        pltpu.sync_copy(pig, oig.at[pl.ds(base, bpw)])
        pltpu.sync_copy(pum, oum.at[pl.ds(base, bpw)])
        pltpu.sync_copy(pim, oim.at[pl.ds(base, bpw)])

    return k(urow, usub, irow, isub, tug, tig, tum, tim)


def _tc_mlp_body(ug_r, ig_r, um_r, im_r, w0a_r, w0b_r, b0_r, w1_r, b1_r,
                 w2_r, b2_r, w3_r, b3_r, wpg_r, wph_r, bp_r, out_r):
    f32 = jnp.float32
    gmf = ug_r[...] * ig_r[...]
    h = jnp.dot(um_r[...], w0a_r[...], preferred_element_type=f32)
    h = h + jnp.dot(im_r[...], w0b_r[...], preferred_element_type=f32)
    h = jnp.maximum(h + b0_r[...], 0.0)
    h = jnp.maximum(jnp.dot(h, w1_r[...], preferred_element_type=f32) + b1_r[...], 0.0)
    h = jnp.maximum(jnp.dot(h, w2_r[...], preferred_element_type=f32) + b2_r[...], 0.0)
    h = jnp.maximum(jnp.dot(h, w3_r[...], preferred_element_type=f32) + b3_r[...], 0.0)
    pred = jnp.dot(gmf, wpg_r[...], preferred_element_type=f32)
    pred = pred + jnp.dot(h, wph_r[...], preferred_element_type=f32)
    out_r[...] = pred + bp_r[...]


def _tc_mlp(ug, ig, um, im, w0a, w0b, b0, w1t, b1, w2t, b2, w3t, b3,
            wpg, wph, bp2):
    nblk = 8
    rb = B // nblk
    row_spec = pl.BlockSpec((rb, D), lambda i: (i, 0))

    def full(x):
        return pl.BlockSpec(x.shape, lambda i: (0,) * x.ndim)

    return pl.pallas_call(
        _tc_mlp_body,
        grid=(nblk,),
        in_specs=[row_spec, row_spec, row_spec, row_spec,
                  full(w0a), full(w0b), full(b0), full(w1t), full(b1),
                  full(w2t), full(b2), full(w3t), full(b3),
                  full(wpg), full(wph), full(bp2)],
        out_specs=pl.BlockSpec((rb, 1), lambda i: (i, 0)),
        out_shape=jax.ShapeDtypeStruct((B, 1), jnp.float32),
    )(ug, ig, um, im, w0a, w0b, b0, w1t, b1, w2t, b2, w3t, b3, wpg, wph, bp2)


def kernel(user_indices, item_indices, user_embed_gmf, item_embed_gmf,
           user_embed_mlp, item_embed_mlp,
           W0, b0, W1, b1, W2, b2, W3, b3, Wp, bp):
    uidx = user_indices.astype(jnp.int32)
    iidx = item_indices.astype(jnp.int32)

    def coords(idx):
        rem = idx % CB
        row = (idx // CB) * PB + (rem % PB)
        sub = ((rem // PB) & 7) << 4
        return row, sub

    urow, usub = coords(uidx)
    irow, isub = coords(iidx)

    tig, tim = _tc_pack2(item_embed_gmf.T, item_embed_mlp.T)
    tug, tum = _tc_pack2(user_embed_gmf.T, user_embed_mlp.T)

    ug, ig, um, im = _sc_gather4(urow, usub, irow, isub, tug, tig, tum, tim)

    w0a = W0[:, :D].T
    w0b = W0[:, D:].T
    wpg = Wp[:, :D].T
    wph = Wp[:, D:].T
    pred = _tc_mlp(ug, ig, um, im, w0a, w0b, b0.reshape(1, -1),
                   W1.T, b1.reshape(1, -1), W2.T, b2.reshape(1, -1),
                   W3.T, b3.reshape(1, -1), wpg, wph, bp.reshape(1, 1))
    return jnp.squeeze(pred, axis=-1)


# speedup vs baseline: 5.3260x; 1.0532x over previous
import functools

import jax
import jax.numpy as jnp
from jax import lax
from jax.experimental import pallas as pl
from jax.experimental.pallas import tpu as pltpu
from jax.experimental.pallas import tpu_sc as plsc

B = 16384
D = 16
CH = 128


CB = 8192
PB = CB // 8


def _tc_pack2(tab_a, tab_b):
    n = tab_a.shape[1]
    grid = (n + CB - 1) // CB

    def body(ina_r, inb_r, outa_r, outb_r):
        for in_r, out_r in ((ina_r, outa_r), (inb_r, outb_r)):
            x = in_r[...]
            stacked = jnp.concatenate(
                [x[:, k * PB:(k + 1) * PB] for k in range(8)], axis=0)
            out_r[...] = stacked.T

    spec_in = pl.BlockSpec((D, CB), lambda i: (0, i))
    spec_out = pl.BlockSpec((PB, 128), lambda i: (i, 0))
    return pl.pallas_call(
        body,
        grid=(grid,),
        in_specs=[spec_in, spec_in],
        out_specs=[spec_out, spec_out],
        out_shape=[jax.ShapeDtypeStruct((grid * PB, 128), jnp.float32)] * 2,
    )(tab_a, tab_b)


def _sc_gather_pair(row, sub, tab1, tab2):
    info = plsc.get_sparse_core_info()
    nw = info.num_cores * info.num_subcores
    bpw = B // nw
    nch = bpw // CH
    mesh = plsc.VectorSubcoreMesh(core_axis_name="c", subcore_axis_name="s")

    @functools.partial(
        pl.kernel,
        mesh=mesh,
        out_type=[jax.ShapeDtypeStruct((B, D), jnp.float32)] * 2,
        scratch_types=[
            pltpu.VMEM((bpw,), jnp.int32),
            pltpu.VMEM((bpw,), jnp.int32),
            pltpu.VMEM((CH, 128), jnp.float32),
            pltpu.VMEM((CH, 128), jnp.float32),
            pltpu.VMEM((bpw, D), jnp.float32),
            pltpu.VMEM((bpw, D), jnp.float32),
            pltpu.SemaphoreType.DMA,
            pltpu.SemaphoreType.DMA,
        ],
        compiler_params=pltpu.CompilerParams(
            use_tc_tiling_on_sc=False, needs_layout_passes=False),
    )
    def k(row_h, sub_h, tab1_h, tab2_h, o1, o2, rv, sv, gb0, gb1,
          p1, p2, sem0, sem1):
        wid = lax.axis_index("s") * info.num_cores + lax.axis_index("c")
        base = wid * bpw
        pltpu.sync_copy(row_h.at[pl.ds(base, bpw)], rv)
        pltpu.sync_copy(sub_h.at[pl.ds(base, bpw)], sv)

        steps = [(tab, ov, c)
                 for tab, ov in [(tab1_h, p1), (tab2_h, p2)]
                 for c in range(nch)]

        gbufs = (gb0, gb1)
        sems = (sem0, sem1)
        iota16 = lax.iota(jnp.int32, 16)

        def issue(s):
            tab, _, c = steps[s]
            return pltpu.async_copy(tab.at[rv.at[pl.ds(c * CH, CH)]],
                                    gbufs[s % 2], sems[s % 2])

        def extract(s, cp):
            _, ov, c = steps[s]
            gb = gbufs[s % 2]
            cp.wait()

            def body(g, _):
                offs = sv[pl.ds(c * CH + g * 16, 16)]
                for jj in range(16):
                    lanei = jnp.broadcast_to(offs[jj], (16,)) + iota16
                    rowi = jnp.broadcast_to(g * 16 + jj, (16,))
                    vals = plsc.load_gather(gb, [rowi, lanei])
                    ov[c * CH + g * 16 + jj, :] = vals
                return 0

            lax.fori_loop(0, CH // 16, body, 0)

        cp = issue(0)
        for s in range(len(steps)):
            nxt = issue(s + 1) if s + 1 < len(steps) else None
            extract(s, cp)
            cp = nxt

        pltpu.sync_copy(p1, o1.at[pl.ds(base, bpw)])
        pltpu.sync_copy(p2, o2.at[pl.ds(base, bpw)])

    return k(row, sub, tab1, tab2)


def _tc_mlp_body(ug_r, ig_r, um_r, im_r, w0a_r, w0b_r, b0_r, w1_r, b1_r,
                 w2_r, b2_r, w3_r, b3_r, wpg_r, wph_r, bp_r, out_r):
    f32 = jnp.float32
    gmf = ug_r[...] * ig_r[...]
    h = jnp.dot(um_r[...], w0a_r[...], preferred_element_type=f32)
    h = h + jnp.dot(im_r[...], w0b_r[...], preferred_element_type=f32)
    h = jnp.maximum(h + b0_r[...], 0.0)
    h = jnp.maximum(jnp.dot(h, w1_r[...], preferred_element_type=f32) + b1_r[...], 0.0)
    h = jnp.maximum(jnp.dot(h, w2_r[...], preferred_element_type=f32) + b2_r[...], 0.0)
    h = jnp.maximum(jnp.dot(h, w3_r[...], preferred_element_type=f32) + b3_r[...], 0.0)
    pred = jnp.dot(gmf, wpg_r[...], preferred_element_type=f32)
    pred = pred + jnp.dot(h, wph_r[...], preferred_element_type=f32)
    out_r[...] = pred + bp_r[...]


def _tc_mlp(ug, ig, um, im, w0a, w0b, b0, w1t, b1, w2t, b2, w3t, b3,
            wpg, wph, bp2):
    nblk = 8
    rb = B // nblk
    row_spec = pl.BlockSpec((rb, D), lambda i: (i, 0))

    def full(x):
        return pl.BlockSpec(x.shape, lambda i: (0,) * x.ndim)

    return pl.pallas_call(
        _tc_mlp_body,
        grid=(nblk,),
        in_specs=[row_spec, row_spec, row_spec, row_spec,
                  full(w0a), full(w0b), full(b0), full(w1t), full(b1),
                  full(w2t), full(b2), full(w3t), full(b3),
                  full(wpg), full(wph), full(bp2)],
        out_specs=pl.BlockSpec((rb, 1), lambda i: (i, 0)),
        out_shape=jax.ShapeDtypeStruct((B, 1), jnp.float32),
    )(ug, ig, um, im, w0a, w0b, b0, w1t, b1, w2t, b2, w3t, b3, wpg, wph, bp2)


def kernel(user_indices, item_indices, user_embed_gmf, item_embed_gmf,
           user_embed_mlp, item_embed_mlp,
           W0, b0, W1, b1, W2, b2, W3, b3, Wp, bp):
    uidx = user_indices.astype(jnp.int32)
    iidx = item_indices.astype(jnp.int32)

    def coords(idx):
        rem = idx % CB
        row = (idx // CB) * PB + (rem % PB)
        sub = ((rem // PB) & 7) << 4
        return row, sub

    urow, usub = coords(uidx)
    irow, isub = coords(iidx)

    tig, tim = _tc_pack2(item_embed_gmf.T, item_embed_mlp.T)
    ig, im = _sc_gather_pair(irow, isub, tig, tim)
    tug, tum = _tc_pack2(user_embed_gmf.T, user_embed_mlp.T)
    ug, um = _sc_gather_pair(urow, usub, tug, tum)

    w0a = W0[:, :D].T
    w0b = W0[:, D:].T
    wpg = Wp[:, :D].T
    wph = Wp[:, D:].T
    pred = _tc_mlp(ug, ig, um, im, w0a, w0b, b0.reshape(1, -1),
                   W1.T, b1.reshape(1, -1), W2.T, b2.reshape(1, -1),
                   W3.T, b3.reshape(1, -1), wpg, wph, bp.reshape(1, 1))
    return jnp.squeeze(pred, axis=-1)


# speedup vs baseline: 6.4772x; 1.2161x over previous
import functools

import jax
import jax.numpy as jnp
from jax import lax
from jax.experimental import pallas as pl
from jax.experimental.pallas import tpu as pltpu
from jax.experimental.pallas import tpu_sc as plsc

B = 16384
D = 16
CH = 128


CB = 16384
PB = CB // 8


def _tc_pack2(tab_a, tab_b):
    n = tab_a.shape[1]
    grid = (n + CB - 1) // CB

    def body(ina_r, inb_r, outa_r, outb_r):
        for in_r, out_r in ((ina_r, outa_r), (inb_r, outb_r)):
            x = in_r[...]
            stacked = jnp.concatenate(
                [x[:, k * PB:(k + 1) * PB] for k in range(8)], axis=0)
            out_r[...] = stacked.T

    spec_in = pl.BlockSpec((D, CB), lambda i: (0, i))
    spec_out = pl.BlockSpec((PB, 128), lambda i: (i, 0))
    return pl.pallas_call(
        body,
        grid=(grid,),
        in_specs=[spec_in, spec_in],
        out_specs=[spec_out, spec_out],
        out_shape=[jax.ShapeDtypeStruct((grid * PB, 128), jnp.float32)] * 2,
    )(tab_a, tab_b)


def _sc_gather_pair(row, sub, tab1, tab2):
    info = plsc.get_sparse_core_info()
    nw = info.num_cores * info.num_subcores
    bpw = B // nw
    nch = bpw // CH
    mesh = plsc.VectorSubcoreMesh(core_axis_name="c", subcore_axis_name="s")

    @functools.partial(
        pl.kernel,
        mesh=mesh,
        out_type=[jax.ShapeDtypeStruct((B, D), jnp.float32)] * 2,
        scratch_types=[
            pltpu.VMEM((bpw,), jnp.int32),
            pltpu.VMEM((bpw,), jnp.int32),
            pltpu.VMEM((CH, 128), jnp.float32),
            pltpu.VMEM((CH, 128), jnp.float32),
            pltpu.VMEM((bpw, D), jnp.float32),
            pltpu.VMEM((bpw, D), jnp.float32),
            pltpu.SemaphoreType.DMA,
            pltpu.SemaphoreType.DMA,
        ],
        compiler_params=pltpu.CompilerParams(
            use_tc_tiling_on_sc=False, needs_layout_passes=False),
    )
    def k(row_h, sub_h, tab1_h, tab2_h, o1, o2, rv, sv, gb0, gb1,
          p1, p2, sem0, sem1):
        wid = lax.axis_index("s") * info.num_cores + lax.axis_index("c")
        base = wid * bpw
        pltpu.sync_copy(row_h.at[pl.ds(base, bpw)], rv)
        pltpu.sync_copy(sub_h.at[pl.ds(base, bpw)], sv)

        steps = [(tab, ov, c)
                 for tab, ov in [(tab1_h, p1), (tab2_h, p2)]
                 for c in range(nch)]

        gbufs = (gb0, gb1)
        sems = (sem0, sem1)
        iota16 = lax.iota(jnp.int32, 16)

        def issue(s):
            tab, _, c = steps[s]
            return pltpu.async_copy(tab.at[rv.at[pl.ds(c * CH, CH)]],
                                    gbufs[s % 2], sems[s % 2])

        def extract(s, cp):
            _, ov, c = steps[s]
            gb = gbufs[s % 2]
            cp.wait()

            def body(g, _):
                offs = sv[pl.ds(c * CH + g * 16, 16)]
                for jj in range(16):
                    lanei = jnp.broadcast_to(offs[jj], (16,)) + iota16
                    rowi = jnp.broadcast_to(g * 16 + jj, (16,))
                    vals = plsc.load_gather(gb, [rowi, lanei])
                    ov[c * CH + g * 16 + jj, :] = vals
                return 0

            lax.fori_loop(0, CH // 16, body, 0)

        cp = issue(0)
        for s in range(len(steps)):
            nxt = issue(s + 1) if s + 1 < len(steps) else None
            extract(s, cp)
            cp = nxt

        pltpu.sync_copy(p1, o1.at[pl.ds(base, bpw)])
        pltpu.sync_copy(p2, o2.at[pl.ds(base, bpw)])

    return k(row, sub, tab1, tab2)


def _tc_mlp_body(ug_r, ig_r, um_r, im_r, w0a_r, w0b_r, b0_r, w1_r, b1_r,
                 w2_r, b2_r, w3_r, b3_r, wpg_r, wph_r, bp_r, out_r):
    f32 = jnp.float32
    gmf = ug_r[...] * ig_r[...]
    h = jnp.dot(um_r[...], w0a_r[...], preferred_element_type=f32)
    h = h + jnp.dot(im_r[...], w0b_r[...], preferred_element_type=f32)
    h = jnp.maximum(h + b0_r[...], 0.0)
    h = jnp.maximum(jnp.dot(h, w1_r[...], preferred_element_type=f32) + b1_r[...], 0.0)
    h = jnp.maximum(jnp.dot(h, w2_r[...], preferred_element_type=f32) + b2_r[...], 0.0)
    h = jnp.maximum(jnp.dot(h, w3_r[...], preferred_element_type=f32) + b3_r[...], 0.0)
    pred = jnp.dot(gmf, wpg_r[...], preferred_element_type=f32)
    pred = pred + jnp.dot(h, wph_r[...], preferred_element_type=f32)
    out_r[...] = pred + bp_r[...]


def _tc_mlp(ug, ig, um, im, w0a, w0b, b0, w1t, b1, w2t, b2, w3t, b3,
            wpg, wph, bp2):
    nblk = 8
    rb = B // nblk
    row_spec = pl.BlockSpec((rb, D), lambda i: (i, 0))

    def full(x):
        return pl.BlockSpec(x.shape, lambda i: (0,) * x.ndim)

    return pl.pallas_call(
        _tc_mlp_body,
        grid=(nblk,),
        in_specs=[row_spec, row_spec, row_spec, row_spec,
                  full(w0a), full(w0b), full(b0), full(w1t), full(b1),
                  full(w2t), full(b2), full(w3t), full(b3),
                  full(wpg), full(wph), full(bp2)],
        out_specs=pl.BlockSpec((rb, 1), lambda i: (i, 0)),
        out_shape=jax.ShapeDtypeStruct((B, 1), jnp.float32),
    )(ug, ig, um, im, w0a, w0b, b0, w1t, b1, w2t, b2, w3t, b3, wpg, wph, bp2)


def kernel(user_indices, item_indices, user_embed_gmf, item_embed_gmf,
           user_embed_mlp, item_embed_mlp,
           W0, b0, W1, b1, W2, b2, W3, b3, Wp, bp):
    uidx = user_indices.astype(jnp.int32)
    iidx = item_indices.astype(jnp.int32)

    def coords(idx):
        rem = idx % CB
        row = (idx // CB) * PB + (rem % PB)
        sub = ((rem // PB) & 7) << 4
        return row, sub

    urow, usub = coords(uidx)
    irow, isub = coords(iidx)

    tig, tim = _tc_pack2(item_embed_gmf.T, item_embed_mlp.T)
    ig, im = _sc_gather_pair(irow, isub, tig, tim)
    tug, tum = _tc_pack2(user_embed_gmf.T, user_embed_mlp.T)
    ug, um = _sc_gather_pair(urow, usub, tug, tum)

    w0a = W0[:, :D].T
    w0b = W0[:, D:].T
    wpg = Wp[:, :D].T
    wph = Wp[:, D:].T
    pred = _tc_mlp(ug, ig, um, im, w0a, w0b, b0.reshape(1, -1),
                   W1.T, b1.reshape(1, -1), W2.T, b2.reshape(1, -1),
                   W3.T, b3.reshape(1, -1), wpg, wph, bp.reshape(1, 1))
    return jnp.squeeze(pred, axis=-1)


# speedup vs baseline: 6.9921x; 1.0795x over previous
import functools

import jax
import jax.numpy as jnp
from jax import lax
from jax.experimental import pallas as pl
from jax.experimental.pallas import tpu as pltpu
from jax.experimental.pallas import tpu_sc as plsc

B = 16384
D = 16
CH = 128


CB = 32768
PB = CB // 8


def _tc_pack2(tab_a, tab_b):
    n = tab_a.shape[1]
    grid = (n + CB - 1) // CB

    def body(ina_r, inb_r, outa_r, outb_r):
        for in_r, out_r in ((ina_r, outa_r), (inb_r, outb_r)):
            x = in_r[...]
            stacked = jnp.concatenate(
                [x[:, k * PB:(k + 1) * PB] for k in range(8)], axis=0)
            out_r[...] = stacked.T

    spec_in = pl.BlockSpec((D, CB), lambda i: (0, i))
    spec_out = pl.BlockSpec((PB, 128), lambda i: (i, 0))
    return pl.pallas_call(
        body,
        grid=(grid,),
        in_specs=[spec_in, spec_in],
        out_specs=[spec_out, spec_out],
        out_shape=[jax.ShapeDtypeStruct((grid * PB, 128), jnp.float32)] * 2,
    )(tab_a, tab_b)


def _sc_gather_pair(row, sub, tab1, tab2):
    info = plsc.get_sparse_core_info()
    nw = info.num_cores * info.num_subcores
    bpw = B // nw
    nch = bpw // CH
    mesh = plsc.VectorSubcoreMesh(core_axis_name="c", subcore_axis_name="s")

    @functools.partial(
        pl.kernel,
        mesh=mesh,
        out_type=[jax.ShapeDtypeStruct((B, D), jnp.float32)] * 2,
        scratch_types=[
            pltpu.VMEM((bpw,), jnp.int32),
            pltpu.VMEM((bpw,), jnp.int32),
            pltpu.VMEM((CH, 128), jnp.float32),
            pltpu.VMEM((CH, 128), jnp.float32),
            pltpu.VMEM((bpw, D), jnp.float32),
            pltpu.VMEM((bpw, D), jnp.float32),
            pltpu.SemaphoreType.DMA,
            pltpu.SemaphoreType.DMA,
        ],
        compiler_params=pltpu.CompilerParams(
            use_tc_tiling_on_sc=False, needs_layout_passes=False),
    )
    def k(row_h, sub_h, tab1_h, tab2_h, o1, o2, rv, sv, gb0, gb1,
          p1, p2, sem0, sem1):
        wid = lax.axis_index("s") * info.num_cores + lax.axis_index("c")
        base = wid * bpw
        pltpu.sync_copy(row_h.at[pl.ds(base, bpw)], rv)
        pltpu.sync_copy(sub_h.at[pl.ds(base, bpw)], sv)

        steps = [(tab, ov, c)
                 for tab, ov in [(tab1_h, p1), (tab2_h, p2)]
                 for c in range(nch)]

        gbufs = (gb0, gb1)
        sems = (sem0, sem1)
        iota16 = lax.iota(jnp.int32, 16)

        def issue(s):
            tab, _, c = steps[s]
            return pltpu.async_copy(tab.at[rv.at[pl.ds(c * CH, CH)]],
                                    gbufs[s % 2], sems[s % 2])

        def extract(s, cp):
            _, ov, c = steps[s]
            gb = gbufs[s % 2]
            cp.wait()

            def body(g, _):
                offs = sv[pl.ds(c * CH + g * 16, 16)]
                for jj in range(16):
                    lanei = jnp.broadcast_to(offs[jj], (16,)) + iota16
                    rowi = jnp.broadcast_to(g * 16 + jj, (16,))
                    vals = plsc.load_gather(gb, [rowi, lanei])
                    ov[c * CH + g * 16 + jj, :] = vals
                return 0

            lax.fori_loop(0, CH // 16, body, 0)

        cp = issue(0)
        for s in range(len(steps)):
            nxt = issue(s + 1) if s + 1 < len(steps) else None
            extract(s, cp)
            cp = nxt

        pltpu.sync_copy(p1, o1.at[pl.ds(base, bpw)])
        pltpu.sync_copy(p2, o2.at[pl.ds(base, bpw)])

    return k(row, sub, tab1, tab2)


def _tc_mlp_body(ug_r, ig_r, um_r, im_r, w0a_r, w0b_r, b0_r, w1_r, b1_r,
                 w2_r, b2_r, w3_r, b3_r, wpg_r, wph_r, bp_r, out_r):
    f32 = jnp.float32
    gmf = ug_r[...] * ig_r[...]
    h = jnp.dot(um_r[...], w0a_r[...], preferred_element_type=f32)
    h = h + jnp.dot(im_r[...], w0b_r[...], preferred_element_type=f32)
    h = jnp.maximum(h + b0_r[...], 0.0)
    h = jnp.maximum(jnp.dot(h, w1_r[...], preferred_element_type=f32) + b1_r[...], 0.0)
    h = jnp.maximum(jnp.dot(h, w2_r[...], preferred_element_type=f32) + b2_r[...], 0.0)
    h = jnp.maximum(jnp.dot(h, w3_r[...], preferred_element_type=f32) + b3_r[...], 0.0)
    pred = jnp.dot(gmf, wpg_r[...], preferred_element_type=f32)
    pred = pred + jnp.dot(h, wph_r[...], preferred_element_type=f32)
    out_r[...] = pred + bp_r[...]


def _tc_mlp(ug, ig, um, im, w0a, w0b, b0, w1t, b1, w2t, b2, w3t, b3,
            wpg, wph, bp2):
    nblk = 8
    rb = B // nblk
    row_spec = pl.BlockSpec((rb, D), lambda i: (i, 0))

    def full(x):
        return pl.BlockSpec(x.shape, lambda i: (0,) * x.ndim)

    return pl.pallas_call(
        _tc_mlp_body,
        grid=(nblk,),
        in_specs=[row_spec, row_spec, row_spec, row_spec,
                  full(w0a), full(w0b), full(b0), full(w1t), full(b1),
                  full(w2t), full(b2), full(w3t), full(b3),
                  full(wpg), full(wph), full(bp2)],
        out_specs=pl.BlockSpec((rb, 1), lambda i: (i, 0)),
        out_shape=jax.ShapeDtypeStruct((B, 1), jnp.float32),
    )(ug, ig, um, im, w0a, w0b, b0, w1t, b1, w2t, b2, w3t, b3, wpg, wph, bp2)


def kernel(user_indices, item_indices, user_embed_gmf, item_embed_gmf,
           user_embed_mlp, item_embed_mlp,
           W0, b0, W1, b1, W2, b2, W3, b3, Wp, bp):
    uidx = user_indices.astype(jnp.int32)
    iidx = item_indices.astype(jnp.int32)

    def coords(idx):
        rem = idx % CB
        row = (idx // CB) * PB + (rem % PB)
        sub = ((rem // PB) & 7) << 4
        return row, sub

    urow, usub = coords(uidx)
    irow, isub = coords(iidx)

    tig, tim = _tc_pack2(item_embed_gmf.T, item_embed_mlp.T)
    ig, im = _sc_gather_pair(irow, isub, tig, tim)
    tug, tum = _tc_pack2(user_embed_gmf.T, user_embed_mlp.T)
    ug, um = _sc_gather_pair(urow, usub, tug, tum)

    w0a = W0[:, :D].T
    w0b = W0[:, D:].T
    wpg = Wp[:, :D].T
    wph = Wp[:, D:].T
    pred = _tc_mlp(ug, ig, um, im, w0a, w0b, b0.reshape(1, -1),
                   W1.T, b1.reshape(1, -1), W2.T, b2.reshape(1, -1),
                   W3.T, b3.reshape(1, -1), wpg, wph, bp.reshape(1, 1))
    return jnp.squeeze(pred, axis=-1)


# speedup vs baseline: 7.0948x; 1.0147x over previous
import functools

import jax
import jax.numpy as jnp
from jax import lax
from jax.experimental import pallas as pl
from jax.experimental.pallas import tpu as pltpu
from jax.experimental.pallas import tpu_sc as plsc

B = 16384
D = 16
CH = 128


CB = 65536
PB = CB // 8


def _tc_pack2(tab_a, tab_b):
    n = tab_a.shape[1]
    grid = (n + CB - 1) // CB

    def body(ina_r, inb_r, outa_r, outb_r):
        for in_r, out_r in ((ina_r, outa_r), (inb_r, outb_r)):
            x = in_r[...]
            stacked = jnp.concatenate(
                [x[:, k * PB:(k + 1) * PB] for k in range(8)], axis=0)
            out_r[...] = stacked.T

    spec_in = pl.BlockSpec((D, CB), lambda i: (0, i))
    spec_out = pl.BlockSpec((PB, 128), lambda i: (i, 0))
    return pl.pallas_call(
        body,
        grid=(grid,),
        in_specs=[spec_in, spec_in],
        out_specs=[spec_out, spec_out],
        out_shape=[jax.ShapeDtypeStruct((grid * PB, 128), jnp.float32)] * 2,
    )(tab_a, tab_b)


def _sc_gather_pair(row, sub, tab1, tab2):
    info = plsc.get_sparse_core_info()
    nw = info.num_cores * info.num_subcores
    bpw = B // nw
    nch = bpw // CH
    mesh = plsc.VectorSubcoreMesh(core_axis_name="c", subcore_axis_name="s")

    @functools.partial(
        pl.kernel,
        mesh=mesh,
        out_type=[jax.ShapeDtypeStruct((B, D), jnp.float32)] * 2,
        scratch_types=[
            pltpu.VMEM((bpw,), jnp.int32),
            pltpu.VMEM((bpw,), jnp.int32),
            pltpu.VMEM((CH, 128), jnp.float32),
            pltpu.VMEM((CH, 128), jnp.float32),
            pltpu.VMEM((bpw, D), jnp.float32),
            pltpu.VMEM((bpw, D), jnp.float32),
            pltpu.SemaphoreType.DMA,
            pltpu.SemaphoreType.DMA,
        ],
        compiler_params=pltpu.CompilerParams(
            use_tc_tiling_on_sc=False, needs_layout_passes=False),
    )
    def k(row_h, sub_h, tab1_h, tab2_h, o1, o2, rv, sv, gb0, gb1,
          p1, p2, sem0, sem1):
        wid = lax.axis_index("s") * info.num_cores + lax.axis_index("c")
        base = wid * bpw
        pltpu.sync_copy(row_h.at[pl.ds(base, bpw)], rv)
        pltpu.sync_copy(sub_h.at[pl.ds(base, bpw)], sv)

        steps = [(tab, ov, c)
                 for tab, ov in [(tab1_h, p1), (tab2_h, p2)]
                 for c in range(nch)]

        gbufs = (gb0, gb1)
        sems = (sem0, sem1)
        iota16 = lax.iota(jnp.int32, 16)

        def issue(s):
            tab, _, c = steps[s]
            return pltpu.async_copy(tab.at[rv.at[pl.ds(c * CH, CH)]],
                                    gbufs[s % 2], sems[s % 2])

        def extract(s, cp):
            _, ov, c = steps[s]
            gb = gbufs[s % 2]
            cp.wait()

            def body(g, _):
                offs = sv[pl.ds(c * CH + g * 16, 16)]
                for jj in range(16):
                    lanei = jnp.broadcast_to(offs[jj], (16,)) + iota16
                    rowi = jnp.broadcast_to(g * 16 + jj, (16,))
                    vals = plsc.load_gather(gb, [rowi, lanei])
                    ov[c * CH + g * 16 + jj, :] = vals
                return 0

            lax.fori_loop(0, CH // 16, body, 0)

        cp = issue(0)
        for s in range(len(steps)):
            nxt = issue(s + 1) if s + 1 < len(steps) else None
            extract(s, cp)
            cp = nxt

        pltpu.sync_copy(p1, o1.at[pl.ds(base, bpw)])
        pltpu.sync_copy(p2, o2.at[pl.ds(base, bpw)])

    return k(row, sub, tab1, tab2)


def _tc_mlp_body(ug_r, ig_r, um_r, im_r, w0a_r, w0b_r, b0_r, w1_r, b1_r,
                 w2_r, b2_r, w3_r, b3_r, wpg_r, wph_r, bp_r, out_r):
    f32 = jnp.float32
    gmf = ug_r[...] * ig_r[...]
    h = jnp.dot(um_r[...], w0a_r[...], preferred_element_type=f32)
    h = h + jnp.dot(im_r[...], w0b_r[...], preferred_element_type=f32)
    h = jnp.maximum(h + b0_r[...], 0.0)
    h = jnp.maximum(jnp.dot(h, w1_r[...], preferred_element_type=f32) + b1_r[...], 0.0)
    h = jnp.maximum(jnp.dot(h, w2_r[...], preferred_element_type=f32) + b2_r[...], 0.0)
    h = jnp.maximum(jnp.dot(h, w3_r[...], preferred_element_type=f32) + b3_r[...], 0.0)
    pred = jnp.dot(gmf, wpg_r[...], preferred_element_type=f32)
    pred = pred + jnp.dot(h, wph_r[...], preferred_element_type=f32)
    out_r[...] = pred + bp_r[...]


def _tc_mlp(ug, ig, um, im, w0a, w0b, b0, w1t, b1, w2t, b2, w3t, b3,
            wpg, wph, bp2):
    nblk = 8
    rb = B // nblk
    row_spec = pl.BlockSpec((rb, D), lambda i: (i, 0))

    def full(x):
        return pl.BlockSpec(x.shape, lambda i: (0,) * x.ndim)

    return pl.pallas_call(
        _tc_mlp_body,
        grid=(nblk,),
        in_specs=[row_spec, row_spec, row_spec, row_spec,
                  full(w0a), full(w0b), full(b0), full(w1t), full(b1),
                  full(w2t), full(b2), full(w3t), full(b3),
                  full(wpg), full(wph), full(bp2)],
        out_specs=pl.BlockSpec((rb, 1), lambda i: (i, 0)),
        out_shape=jax.ShapeDtypeStruct((B, 1), jnp.float32),
    )(ug, ig, um, im, w0a, w0b, b0, w1t, b1, w2t, b2, w3t, b3, wpg, wph, bp2)


def kernel(user_indices, item_indices, user_embed_gmf, item_embed_gmf,
           user_embed_mlp, item_embed_mlp,
           W0, b0, W1, b1, W2, b2, W3, b3, Wp, bp):
    uidx = user_indices.astype(jnp.int32)
    iidx = item_indices.astype(jnp.int32)

    def coords(idx):
        rem = idx % CB
        row = (idx // CB) * PB + (rem % PB)
        sub = ((rem // PB) & 7) << 4
        return row, sub

    urow, usub = coords(uidx)
    irow, isub = coords(iidx)

    tig, tim = _tc_pack2(item_embed_gmf.T, item_embed_mlp.T)
    ig, im = _sc_gather_pair(irow, isub, tig, tim)
    tug, tum = _tc_pack2(user_embed_gmf.T, user_embed_mlp.T)
    ug, um = _sc_gather_pair(urow, usub, tug, tum)

    w0a = W0[:, :D].T
    w0b = W0[:, D:].T
    wpg = Wp[:, :D].T
    wph = Wp[:, D:].T
    pred = _tc_mlp(ug, ig, um, im, w0a, w0b, b0.reshape(1, -1),
                   W1.T, b1.reshape(1, -1), W2.T, b2.reshape(1, -1),
                   W3.T, b3.reshape(1, -1), wpg, wph, bp.reshape(1, 1))
    return jnp.squeeze(pred, axis=-1)
